# baseline (speedup 1.0000x reference)
import jax
import jax.numpy as jnp
from jax.experimental import pallas as pl

_XGRID = 10
_EGRID = 40


def _copy_body(a_ref, o_ref):
    o_ref[...] = a_ref[...]


def _copy(a, grid):
    rb = a.shape[0] // grid
    return pl.pallas_call(
        _copy_body,
        grid=(grid,),
        in_specs=[pl.BlockSpec((rb, a.shape[1]), lambda i: (i, 0))],
        out_specs=pl.BlockSpec((rb, a.shape[1]), lambda i: (i, 0)),
        out_shape=jax.ShapeDtypeStruct(a.shape, a.dtype),
    )(a)


def kernel(x, edge_index, edge_attr):
    del edge_index
    return (_copy(x, _XGRID), _copy(edge_attr, _EGRID))


# speedup vs baseline: 1.4753x; 1.4753x over previous
import functools

import jax
import jax.numpy as jnp
from jax import lax
from jax.experimental import pallas as pl
from jax.experimental.pallas import tpu as pltpu
from jax.experimental.pallas import tpu_sc as plsc

_INFO = plsc.get_sparse_core_info()
_NC, _NS = _INFO.num_cores, _INFO.num_subcores
_NW = _NC * _NS


def _sc_copy(x):
    n_rows, d = x.shape
    rows_w = n_rows // _NW
    tail = n_rows - rows_w * _NW
    mesh = plsc.VectorSubcoreMesh(core_axis_name="c", subcore_axis_name="s")

    @functools.partial(
        pl.kernel,
        mesh=mesh,
        out_type=jax.ShapeDtypeStruct((n_rows, d), x.dtype),
        scratch_types=[pltpu.SemaphoreType.DMA],
    )
    def k(x_hbm, out_hbm, sem):
        wid = lax.axis_index("s") * _NC + lax.axis_index("c")
        base = wid * rows_w
        pltpu.async_copy(
            x_hbm.at[pl.ds(base, rows_w)], out_hbm.at[pl.ds(base, rows_w)], sem
        ).wait()
        if tail:
            @pl.when(wid == 0)
            def _():
                tb = rows_w * _NW
                pltpu.async_copy(
                    x_hbm.at[pl.ds(tb, tail)], out_hbm.at[pl.ds(tb, tail)], sem
                ).wait()

    return k(x)


def kernel(x, edge_index, edge_attr):
    del edge_index
    return (_sc_copy(x), edge_attr)


# speedup vs baseline: 12.0036x; 8.1363x over previous
import jax
import jax.numpy as jnp
from jax.experimental import pallas as pl

_GRID = 10


def _copy_body(x_ref, xo_ref):
    xo_ref[...] = x_ref[...]


def kernel(x, edge_index, edge_attr):
    del edge_index
    n_nodes, d_feat = x.shape
    xb = n_nodes // _GRID
    x_out = pl.pallas_call(
        _copy_body,
        grid=(_GRID,),
        in_specs=[pl.BlockSpec((xb, d_feat), lambda i: (i, 0))],
        out_specs=pl.BlockSpec((xb, d_feat), lambda i: (i, 0)),
        out_shape=jax.ShapeDtypeStruct(x.shape, x.dtype),
    )(x)
    return (x_out, edge_attr)
